# baseline (speedup 1.0000x reference)
import functools

import jax
import jax.numpy as jnp
from jax import lax
from jax.experimental import pallas as pl
from jax.experimental.pallas import tpu as pltpu
from jax.experimental.pallas import tpu_sc as plsc

B = 16384
L_ROWS = 100
D = 1024
N_ENC = 5

NC = 2
NS = 16
NW = NC * NS
B_PER_W = B // NW
CHUNK = 8
NCHUNK = B_PER_W // CHUNK
LANES = 16
VPR = D // LANES


def _merge_body(tids_ref, router_ref, enc_ref, out_ref):
    t = tids_ref[0]
    acc = router_ref[t, 0] * enc_ref[0]
    for k in range(1, N_ENC):
        acc += router_ref[t, k] * enc_ref[k]
    out_ref[...] = acc


def _merge(tids, router, enc_tables):
    return pl.pallas_call(
        _merge_body,
        out_shape=jax.ShapeDtypeStruct((L_ROWS, D), jnp.float32),
        in_specs=[
            pl.BlockSpec(memory_space=pltpu.SMEM),
            pl.BlockSpec(memory_space=pltpu.SMEM),
            pl.BlockSpec(memory_space=pltpu.VMEM),
        ],
        out_specs=pl.BlockSpec(memory_space=pltpu.VMEM),
    )(tids, router, enc_tables)


@functools.cache
def _make_sc_gather():
    mesh = plsc.VectorSubcoreMesh(
        core_axis_name="c", subcore_axis_name="s", num_cores=NC, num_subcores=NS
    )

    @functools.partial(
        pl.kernel,
        out_type=jax.ShapeDtypeStruct((B, D), jnp.float32),
        mesh=mesh,
        scratch_types=[
            pltpu.VMEM((B_PER_W,), jnp.int32),
            pltpu.VMEM((L_ROWS, D), jnp.float32),
            pltpu.VMEM((CHUNK, D), jnp.float32),
            pltpu.VMEM((CHUNK, D), jnp.float32),
            pltpu.SemaphoreType.DMA,
        ],
    )
    def _sc_gather(idx_hbm, rw_hbm, out_hbm, idx_v, table_v, sbuf_a, sbuf_b,
                   osem):
        wid = lax.axis_index("s") * NC + lax.axis_index("c")
        base = wid * B_PER_W
        pltpu.sync_copy(idx_hbm.at[wid], idx_v)
        pltpu.sync_copy(rw_hbm, table_v)

        sbufs = (sbuf_a, sbuf_b)

        def body(g, carry):
            ivec = idx_v[pl.ds(g * 2 * CHUNK, LANES)]
            for p in range(2):
                sbuf = sbufs[p]

                @pl.when(g > 0)
                def _drain():
                    pltpu.make_async_copy(
                        sbuf, out_hbm.at[pl.ds(base, CHUNK)], osem
                    ).wait()

                for j in range(CHUNK):
                    row = ivec[p * CHUNK + j]
                    for k in range(VPR):
                        sl = pl.ds(k * LANES, LANES)
                        sbuf[j, sl] = table_v[row, sl]
                pltpu.async_copy(
                    sbuf,
                    out_hbm.at[pl.ds(base + (2 * g + p) * CHUNK, CHUNK)],
                    osem,
                )
            return carry

        lax.fori_loop(0, NCHUNK // 2, body, 0)
        for sbuf in sbufs:
            pltpu.make_async_copy(
                sbuf, out_hbm.at[pl.ds(base, CHUNK)], osem
            ).wait()

    return _sc_gather


def kernel(prompt_token_ids, tids, router, enc_tables, input_ids):
    del input_ids
    rw = _merge(tids, router, enc_tables)
    idx = prompt_token_ids.astype(jnp.int32).reshape(NW, B_PER_W)
    return _make_sc_gather()(idx, rw)


# speedup vs baseline: 1.5307x; 1.5307x over previous
import functools

import jax
import jax.numpy as jnp
from jax import lax
from jax.experimental import pallas as pl
from jax.experimental.pallas import tpu as pltpu
from jax.experimental.pallas import tpu_sc as plsc

B = 16384
L_ROWS = 100
D = 1024
N_ENC = 5

NC = 2
NS = 16
NW = NC * NS
B_PER_W = B // NW
CHUNK = 32
NCHUNK = B_PER_W // CHUNK
LANES = 16
VPR = D // LANES
MROWS = 8


@functools.cache
def _make_sc_kernel():
    mesh = plsc.VectorSubcoreMesh(
        core_axis_name="c", subcore_axis_name="s", num_cores=NC, num_subcores=NS
    )

    @functools.partial(
        pl.kernel,
        out_type=(
            jax.ShapeDtypeStruct((B, D), jnp.float32),
            jax.ShapeDtypeStruct((L_ROWS, D), jnp.float32),
        ),
        mesh=mesh,
        scratch_types=[
            pltpu.VMEM((LANES,), jnp.int32),
            pltpu.VMEM((NC, LANES), jnp.float32),
            pltpu.VMEM((N_ENC, MROWS, D), jnp.float32),
            pltpu.VMEM((MROWS, D), jnp.float32),
            pltpu.VMEM((NCHUNK, CHUNK), jnp.int32),
            pltpu.VMEM((CHUNK, D), jnp.float32),
            pltpu.VMEM((CHUNK, D), jnp.float32),
            pltpu.SemaphoreType.DMA,
            pltpu.SemaphoreType.DMA,
            pltpu.SemaphoreType.DMA,
        ],
    )
    def _sc_kernel(idx_hbm, tids_hbm, router_hbm, enc_hbm, out_hbm, rw_hbm,
                   tids_v, router_v, tbl_v, rw_v, idx_v, rows_a, rows_b,
                   gsem_a, gsem_b, osem):
        sid = lax.axis_index("s")
        wid = sid * NC + lax.axis_index("c")
        base = wid * B_PER_W

        pltpu.sync_copy(idx_hbm.at[wid], idx_v)
        pltpu.sync_copy(tids_hbm, tids_v)
        pltpu.sync_copy(router_hbm, router_v)
        t = tids_v[...][0]
        rvec = router_v[t, :]

        @pl.when(sid < 12)
        def _full():
            row0 = sid * MROWS
            for k in range(N_ENC):
                pltpu.sync_copy(enc_hbm.at[k, pl.ds(row0, MROWS), :],
                                tbl_v.at[k])

            def mbody(cc, carry):
                sl = pl.ds(cc * LANES, LANES)
                for rr in range(MROWS):
                    acc = rvec[0] * tbl_v[0, rr, sl]
                    for k in range(1, N_ENC):
                        acc += rvec[k] * tbl_v[k, rr, sl]
                    rw_v[rr, sl] = acc
                return carry

            lax.fori_loop(0, VPR, mbody, 0)
            pltpu.sync_copy(rw_v, rw_hbm.at[pl.ds(row0, MROWS)])

        @pl.when(sid == 12)
        def _tail():
            for k in range(N_ENC):
                pltpu.sync_copy(enc_hbm.at[k, pl.ds(96, 4), :],
                                tbl_v.at[k, pl.ds(0, 4)])

            def mbody(cc, carry):
                sl = pl.ds(cc * LANES, LANES)
                for rr in range(4):
                    acc = rvec[0] * tbl_v[0, rr, sl]
                    for k in range(1, N_ENC):
                        acc += rvec[k] * tbl_v[k, rr, sl]
                    rw_v[rr, sl] = acc
                return carry

            lax.fori_loop(0, VPR, mbody, 0)
            pltpu.sync_copy(rw_v.at[pl.ds(0, 4)], rw_hbm.at[pl.ds(96, 4)])

        plsc.subcore_barrier()

        bufs = ((rows_a, gsem_a), (rows_b, gsem_b))

        def body(g, carry):
            for p, (rows, gsem) in enumerate(bufs):
                c = 2 * g + p

                @pl.when(g > 0)
                def _drain():
                    pltpu.make_async_copy(
                        rows, out_hbm.at[pl.ds(base, CHUNK)], osem
                    ).wait()

                pltpu.async_copy(rw_hbm.at[idx_v.at[c]], rows, gsem).wait()
                pltpu.async_copy(
                    rows, out_hbm.at[pl.ds(base + c * CHUNK, CHUNK)], osem
                )
            return carry

        lax.fori_loop(0, NCHUNK // 2, body, 0)
        for rows, _ in bufs:
            pltpu.make_async_copy(
                rows, out_hbm.at[pl.ds(base, CHUNK)], osem
            ).wait()

    return _sc_kernel


def kernel(prompt_token_ids, tids, router, enc_tables, input_ids):
    del input_ids
    idx = prompt_token_ids.astype(jnp.int32).reshape(NW, NCHUNK, CHUNK)
    tids_pad = jnp.pad(tids.astype(jnp.int32), (0, LANES - tids.shape[0]))
    router_pad = jnp.pad(router, ((0, 0), (0, LANES - router.shape[1])))
    out, _ = _make_sc_kernel()(idx, tids_pad, router_pad, enc_tables)
    return out
